# Initial kernel scaffold; baseline (speedup 1.0000x reference)
import jax
import jax.numpy as jnp
from jax.experimental import pallas as pl


def kernel(h, x, params, mask_ligand, edge_index):
    raise NotImplementedError("write your pallas kernel here")


# speedup vs baseline: 11.7520x; 11.7520x over previous
import functools

import jax
import jax.numpy as jnp
import numpy as np
from jax import lax
from jax.experimental import pallas as pl
from jax.experimental.pallas import tpu as pltpu
from jax.experimental.pallas import tpu_sc as plsc

_T = 1024
_W = 512
_C = 512
_NH = 16
_HD = 8
_NG = 20
_RMAX = 10.0
_F32 = jnp.float32


def _sc_gather(table, idx):
    V, D = table.shape
    B = idx.shape[0]
    info = plsc.get_sparse_core_info()
    NW = info.num_cores * info.num_subcores
    b_per_w = B // NW
    n_chunks = b_per_w // _C
    mesh = plsc.VectorSubcoreMesh(core_axis_name="c", subcore_axis_name="s")

    @functools.partial(
        pl.kernel, mesh=mesh,
        out_type=jax.ShapeDtypeStruct((B, D), _F32),
        scratch_types=[
            pltpu.VMEM((_C,), jnp.int32),
            pltpu.VMEM((_C, D), _F32),
            pltpu.SemaphoreType.DMA,
        ],
    )
    def gather_k(table_hbm, idx_hbm, out_hbm, idx_v, rows_v, sem):
        wid = lax.axis_index("s") * info.num_cores + lax.axis_index("c")
        base = wid * b_per_w
        for j in range(n_chunks):
            off = base + j * _C
            pltpu.sync_copy(idx_hbm.at[pl.ds(off, _C)], idx_v)
            pltpu.async_copy(table_hbm.at[idx_v], rows_v, sem).wait()
            pltpu.sync_copy(rows_v, out_hbm.at[pl.ds(off, _C)])

    return gather_k(table, idx)


def _ln_relu(y, g, be):
    mu = jnp.mean(y, axis=-1, keepdims=True)
    d = y - mu
    var = jnp.mean(d * d, axis=-1, keepdims=True)
    y = d * lax.rsqrt(var + 1e-5) * g + be
    return jnp.maximum(y, 0.0)


def _head_sum_mat():
    r = lax.broadcasted_iota(jnp.int32, (128, _NH), 0) // _HD
    c = lax.broadcasted_iota(jnp.int32, (128, _NH), 1)
    return (r == c).astype(_F32)


def _head_rep_mat():
    r = lax.broadcasted_iota(jnp.int32, (_NH, 128), 0)
    c = lax.broadcasted_iota(jnp.int32, (_NH, 128), 1) // _HD
    return (r == c).astype(_F32)


def _r_feat(xms, xmd):
    col = lax.broadcasted_iota(jnp.int32, (1, 16), 1)
    relm = (xmd - xms) * (col < 3).astype(_F32)
    dist2 = jnp.sum(relm * relm, axis=-1, keepdims=True)
    dist = jnp.sqrt(dist2 + 1e-12)
    offs = lax.broadcasted_iota(jnp.int32, (1, _NG), 1).astype(_F32) * (
        _RMAX / (_NG - 1))
    coeff = -0.5 / float(_RMAX / (_NG - 1)) ** 2
    df = jnp.exp(coeff * (dist - offs) ** 2)
    mls = xms[:, 3:4]
    mld = xmd[:, 3:4]
    cls = (1.0 - mls) * 2.0 + (1.0 - mld)
    r80 = jnp.concatenate(
        [df * (cls == float(c)).astype(_F32) for c in range(4)], axis=1)
    return r80, relm


def _dotT(a, b):
    return lax.dot_general(a, b, (((0,), (0,)), ((), ())),
                           preferred_element_type=_F32)


def _mm(a, b):
    return jnp.dot(a, b, preferred_element_type=_F32)


def _qmlp_body(h_ref, W1, b1, g, be, W2, b2, q_ref):
    y = _ln_relu(_mm(h_ref[...], W1[...]) + b1[...], g[...], be[...])
    q_ref[...] = _mm(y, W2[...]) + b2[...]


def _h_update_body(num_ref, den_ref, h_ref,
                   oW1o, oW1h, ob1, og, obe, oW2, ob2,
                   qW1, qb1, qg, qbe, qW2, qb2,
                   hout_ref, q2_ref):
    St = _head_rep_mat()
    h = h_ref[...]
    att = num_ref[...] / (_mm(den_ref[...], St) + 1e-16)
    y = _mm(att, oW1o[...]) + _mm(h, oW1h[...]) + ob1[...]
    y = _ln_relu(y, og[...], obe[...])
    hout = _mm(y, oW2[...]) + ob2[...] + h
    hout_ref[...] = hout
    y2 = _ln_relu(_mm(hout, qW1[...]) + qb1[...], qg[...], qbe[...])
    q2_ref[...] = _mm(y2, qW2[...]) + qb2[...]


def _x_update_body(xm_ref, num2_ref, den2_ref, xout_ref):
    xm = xm_ref[...]
    den = den2_ref[...] + 1e-16
    num2 = num2_ref[...]
    cols = []
    for c in range(3):
        ratio = num2[:, c * _NH:(c + 1) * _NH] / den
        d = jnp.sum(ratio, axis=-1, keepdims=True) * (1.0 / _NH)
        cols.append(xm[:, c:c + 1] + d * xm[:, 3:4])
    cols.append(jnp.zeros((xm.shape[0], 13), _F32))
    xout_ref[...] = jnp.concatenate(cols, axis=1)


def _edge1_body(b0_ref, nw_ref, dst_ref, hs_ref, hd_ref, qd_ref,
                xms_ref, xmd_ref,
                kW1r, kW1d, kW1s, kb1, kg, kbe, kW2, kb2,
                vW1r, vW1d, vW1s, vb1, vg, vbe, vW2, vb2,
                ewW, ewb, num_ref, den_ref, *, E_static):
    i = pl.program_id(0)

    @pl.when(i == 0)
    def _init():
        num_ref[...] = jnp.zeros_like(num_ref)
        den_ref[...] = jnp.zeros_like(den_ref)

    dst = dst_ref[0, 0, :]
    hs = hs_ref[...]
    hd = hd_ref[...]
    qd = qd_ref[...]
    r80, _ = _r_feat(xms_ref[...][:, :16], xmd_ref[...][:, :16])

    yk = _mm(r80, kW1r[...]) + _mm(hd, kW1d[...]) + _mm(hs, kW1s[...]) + kb1[...]
    k = _mm(_ln_relu(yk, kg[...], kbe[...]), kW2[...]) + kb2[...]
    yv = _mm(r80, vW1r[...]) + _mm(hd, vW1d[...]) + _mm(hs, vW1s[...]) + vb1[...]
    v = _mm(_ln_relu(yv, vg[...], vbe[...]), vW2[...]) + vb2[...]
    v = v * jax.nn.sigmoid(_mm(r80, ewW[...]) + ewb[...])

    S = _head_sum_mat()
    l = _mm(qd * k, S) * (1.0 / np.sqrt(_HD))
    gidx = i * _T + lax.broadcasted_iota(jnp.int32, (_T, 1), 0)
    valid = (gidx < E_static).astype(_F32)
    e = jnp.exp(jnp.clip(l, -50.0, 50.0)) * valid
    ev = _mm(e, _head_rep_mat()) * v

    b0 = b0_ref[i]

    def wbody(w, carry):
        base = b0 + w * _W
        oh = (dst[:, None] ==
              base + lax.broadcasted_iota(jnp.int32, (_T, _W), 1)).astype(_F32)
        num_ref[pl.ds(base, _W), :] += _dotT(oh, ev)
        den_ref[pl.ds(base, _W), :] += _dotT(oh, e)
        return carry

    lax.fori_loop(0, nw_ref[i], wbody, 0)


def _edge2_body(b0_ref, nw_ref, dst_ref, hs_ref, hd_ref, qd_ref,
                xms_ref, xmd_ref,
                kW1r, kW1d, kW1s, kb1, kg, kbe, kW2, kb2,
                vW1r, vW1d, vW1s, vb1, vg, vbe, vW2, vb2,
                ewW, ewb, num_ref, den_ref, *, E_static):
    i = pl.program_id(0)

    @pl.when(i == 0)
    def _init():
        num_ref[...] = jnp.zeros_like(num_ref)
        den_ref[...] = jnp.zeros_like(den_ref)

    dst = dst_ref[0, 0, :]
    hs = hs_ref[...]
    hd = hd_ref[...]
    qd = qd_ref[...]
    r80, relm = _r_feat(xms_ref[...][:, :16], xmd_ref[...][:, :16])

    yk = _mm(r80, kW1r[...]) + _mm(hd, kW1d[...]) + _mm(hs, kW1s[...]) + kb1[...]
    k = _mm(_ln_relu(yk, kg[...], kbe[...]), kW2[...]) + kb2[...]
    yv = _mm(r80, vW1r[...]) + _mm(hd, vW1d[...]) + _mm(hs, vW1s[...]) + vb1[...]
    xv = _mm(_ln_relu(yv, vg[...], vbe[...]), vW2[...]) + vb2[...]
    xv = xv * jax.nn.sigmoid(_mm(r80, ewW[...]) + ewb[...])

    S = _head_sum_mat()
    l = _mm(qd * k, S) * (1.0 / np.sqrt(_HD))
    gidx = i * _T + lax.broadcasted_iota(jnp.int32, (_T, 1), 0)
    valid = (gidx < E_static).astype(_F32)
    e = jnp.exp(jnp.clip(l, -50.0, 50.0)) * valid
    w16 = e * xv
    w48 = jnp.concatenate([w16 * relm[:, c:c + 1] for c in range(3)], axis=1)

    b0 = b0_ref[i]

    def wbody(w, carry):
        base = b0 + w * _W
        oh = (dst[:, None] ==
              base + lax.broadcasted_iota(jnp.int32, (_T, _W), 1)).astype(_F32)
        num_ref[pl.ds(base, _W), :] += _dotT(oh, w48)
        den_ref[pl.ds(base, _W), :] += _dotT(oh, e)
        return carry

    lax.fori_loop(0, nw_ref[i], wbody, 0)


def _full(shape):
    return pl.BlockSpec(shape, lambda i, b0, nw: tuple(0 for _ in shape))


def _edge_call(body, b0, nw, dst3, hs, hd, qd, xms, xmd, wts, E, Npd, dout):
    nT = dst3.shape[0]
    espec = [
        pl.BlockSpec((1, 1, _T), lambda i, b0, nw: (i, 0, 0)),
        pl.BlockSpec((_T, 128), lambda i, b0, nw: (i, 0)),
        pl.BlockSpec((_T, 128), lambda i, b0, nw: (i, 0)),
        pl.BlockSpec((_T, 128), lambda i, b0, nw: (i, 0)),
        pl.BlockSpec((_T, 128), lambda i, b0, nw: (i, 0)),
        pl.BlockSpec((_T, 128), lambda i, b0, nw: (i, 0)),
    ]
    wspec = [_full(w.shape) for w in wts]
    grid_spec = pltpu.PrefetchScalarGridSpec(
        num_scalar_prefetch=2,
        grid=(nT,),
        in_specs=espec + wspec,
        out_specs=[
            pl.BlockSpec((Npd, dout), lambda i, b0, nw: (0, 0)),
            pl.BlockSpec((Npd, 16), lambda i, b0, nw: (0, 0)),
        ],
    )
    return pl.pallas_call(
        functools.partial(body, E_static=E),
        grid_spec=grid_spec,
        out_shape=[
            jax.ShapeDtypeStruct((Npd, dout), _F32),
            jax.ShapeDtypeStruct((Npd, 16), _F32),
        ],
    )(b0, nw, dst3, hs, hd, qd, xms, xmd, *wts)


def _node_call(body, ins, outs_dims, N, Tn):
    specs = [pl.BlockSpec((Tn, a.shape[1]), lambda i: (i, 0)) if a.shape[0] == N
             else pl.BlockSpec(a.shape, lambda i: (0, 0)) for a in ins]
    return pl.pallas_call(
        body,
        grid=(N // Tn,),
        in_specs=specs,
        out_specs=[pl.BlockSpec((Tn, d), lambda i: (i, 0)) for d in outs_dims],
        out_shape=[jax.ShapeDtypeStruct((N, d), _F32) for d in outs_dims],
    )(*ins)


def _mlp_wts(p, split=None):
    W1 = p['W1']
    row = lambda a: a.reshape(1, -1)
    if split is None:
        parts = [W1]
    else:
        cuts = [0] + list(split) + [W1.shape[0]]
        parts = [W1[cuts[j]:cuts[j + 1]] for j in range(len(cuts) - 1)]
    return parts + [row(p['b1']), row(p['g']), row(p['be']), p['W2'],
                    row(p['b2'])]


def kernel(h, x, params, mask_ligand, edge_index):
    N, D = h.shape
    E = edge_index.shape[1]
    Tn = 2000 if N % 2000 == 0 else N
    Npd = N + _W

    order = jnp.argsort(edge_index[1])
    dsts = edge_index[1][order]
    srcs = edge_index[0][order]
    quantum = int(np.lcm(32 * _C, _T))
    E_pad = int(-(-E // quantum) * quantum)
    dsts_p = jnp.pad(dsts, (0, E_pad - E), constant_values=N - 1)
    srcs_p = jnp.pad(srcs, (0, E_pad - E))
    nT = E_pad // _T
    dmat = dsts_p.reshape(nT, _T)
    b0 = dmat[:, 0].astype(jnp.int32)
    nw = ((dmat[:, -1] - dmat[:, 0]) // _W + 1).astype(jnp.int32)
    dst3 = dsts_p.reshape(nT, 1, _T)

    mlf = mask_ligand.astype(_F32)[:, None]
    xm = jnp.concatenate([x, mlf, jnp.zeros((N, 124), _F32)], axis=1)

    q = _node_call(_qmlp_body, [h] + _mlp_wts(params['x2h_q']), [D], N, Tn)[0]
    hs = _sc_gather(h, srcs_p)
    hd = _sc_gather(h, dsts_p)
    qd = _sc_gather(q, dsts_p)
    xms = _sc_gather(xm, srcs_p)
    xmd = _sc_gather(xm, dsts_p)

    ew1 = jnp.broadcast_to(params['x2h_ew_W'], (4 * _NG, 128))
    eb1 = jnp.broadcast_to(params['x2h_ew_b'].reshape(1, 1), (1, 128))
    wts1 = (_mlp_wts(params['x2h_k'], (4 * _NG, 4 * _NG + D))
            + _mlp_wts(params['x2h_v'], (4 * _NG, 4 * _NG + D))
            + [ew1, eb1])
    num, den = _edge_call(_edge1_body, b0, nw, dst3, hs, hd, qd, xms, xmd,
                          wts1, E, Npd, 128)

    h_out, q2 = _node_call(
        _h_update_body,
        [num[:N], den[:N], h]
        + _mlp_wts(params['x2h_out'], (D,))
        + _mlp_wts(params['h2x_q']),
        [D, D], N, Tn)

    hs2 = _sc_gather(h_out, srcs_p)
    hd2 = _sc_gather(h_out, dsts_p)
    qd2 = _sc_gather(q2, dsts_p)

    ew2 = jnp.broadcast_to(params['h2x_ew_W'], (4 * _NG, 16))
    eb2 = jnp.broadcast_to(params['h2x_ew_b'].reshape(1, 1), (1, 16))
    wts2 = (_mlp_wts(params['h2x_k'], (4 * _NG, 4 * _NG + D))
            + _mlp_wts(params['h2x_v'], (4 * _NG, 4 * _NG + D))
            + [ew2, eb2])
    num2, den2 = _edge_call(_edge2_body, b0, nw, dst3, hs2, hd2, qd2, xms, xmd,
                            wts2, E, Npd, 48)

    xout = _node_call(_x_update_body, [xm, num2[:N], den2[:N]], [16], N, Tn)[0]
    return h_out, xout[:, :3]


# speedup vs baseline: 16.7840x; 1.4282x over previous
import functools

import jax
import jax.numpy as jnp
import numpy as np
from jax import lax
from jax.experimental import pallas as pl
from jax.experimental.pallas import tpu as pltpu
from jax.experimental.pallas import tpu_sc as plsc

_T = 1024
_W = 512
_C = 512
_NH = 16
_HD = 8
_NG = 20
_RMAX = 10.0
_F32 = jnp.float32


def _sc_gather(table, idx):
    V, D = table.shape
    B = idx.shape[0]
    info = plsc.get_sparse_core_info()
    NW = info.num_cores * info.num_subcores
    b_per_w = B // NW
    n_chunks = b_per_w // _C
    mesh = plsc.VectorSubcoreMesh(core_axis_name="c", subcore_axis_name="s")

    @functools.partial(
        pl.kernel, mesh=mesh,
        out_type=jax.ShapeDtypeStruct((B, D), _F32),
        scratch_types=[
            pltpu.VMEM((_C,), jnp.int32),
            pltpu.VMEM((_C, D), _F32),
            pltpu.SemaphoreType.DMA,
        ],
    )
    def gather_k(table_hbm, idx_hbm, out_hbm, idx_v, rows_v, sem):
        wid = lax.axis_index("s") * info.num_cores + lax.axis_index("c")
        base = wid * b_per_w
        for j in range(n_chunks):
            off = base + j * _C
            pltpu.sync_copy(idx_hbm.at[pl.ds(off, _C)], idx_v)
            pltpu.async_copy(table_hbm.at[idx_v], rows_v, sem).wait()
            pltpu.sync_copy(rows_v, out_hbm.at[pl.ds(off, _C)])

    return gather_k(table, idx)


def _ln_relu(y, g, be):
    mu = jnp.mean(y, axis=-1, keepdims=True)
    d = y - mu
    var = jnp.mean(d * d, axis=-1, keepdims=True)
    y = d * lax.rsqrt(var + 1e-5) * g + be
    return jnp.maximum(y, 0.0)


def _head_sum_mat():
    r = lax.broadcasted_iota(jnp.int32, (128, _NH), 0) // _HD
    c = lax.broadcasted_iota(jnp.int32, (128, _NH), 1)
    return (r == c).astype(_F32)


def _head_rep_mat():
    r = lax.broadcasted_iota(jnp.int32, (_NH, 128), 0)
    c = lax.broadcasted_iota(jnp.int32, (_NH, 128), 1) // _HD
    return (r == c).astype(_F32)


def _r_feat(xms, xmd):
    col = lax.broadcasted_iota(jnp.int32, (1, 16), 1)
    relm = (xmd - xms) * (col < 3).astype(_F32)
    dist2 = jnp.sum(relm * relm, axis=-1, keepdims=True)
    dist = jnp.sqrt(dist2 + 1e-12)
    offs = lax.broadcasted_iota(jnp.int32, (1, _NG), 1).astype(_F32) * (
        _RMAX / (_NG - 1))
    coeff = -0.5 / float(_RMAX / (_NG - 1)) ** 2
    df = jnp.exp(coeff * (dist - offs) ** 2)
    mls = xms[:, 3:4]
    mld = xmd[:, 3:4]
    cls = (1.0 - mls) * 2.0 + (1.0 - mld)
    r80 = jnp.concatenate(
        [df * (cls == float(c)).astype(_F32) for c in range(4)], axis=1)
    return r80, relm


def _dotT(a, b):
    return lax.dot_general(a, b, (((0,), (0,)), ((), ())),
                           preferred_element_type=_F32)


def _mm(a, b):
    return jnp.dot(a, b, preferred_element_type=_F32)


def _qmlp_body(h_ref, W1, b1, g, be, W2, b2, q_ref):
    y = _ln_relu(_mm(h_ref[...], W1[...]) + b1[...], g[...], be[...])
    q_ref[...] = _mm(y, W2[...]) + b2[...]


def _h_update_body(num_ref, den_ref, h_ref,
                   oW1o, oW1h, ob1, og, obe, oW2, ob2,
                   qW1, qb1, qg, qbe, qW2, qb2,
                   hout_ref, q2_ref):
    St = _head_rep_mat()
    h = h_ref[...]
    att = num_ref[...] / (_mm(den_ref[...], St) + 1e-16)
    y = _mm(att, oW1o[...]) + _mm(h, oW1h[...]) + ob1[...]
    y = _ln_relu(y, og[...], obe[...])
    hout = _mm(y, oW2[...]) + ob2[...] + h
    hout_ref[...] = hout
    y2 = _ln_relu(_mm(hout, qW1[...]) + qb1[...], qg[...], qbe[...])
    q2_ref[...] = _mm(y2, qW2[...]) + qb2[...]


def _x_update_body(xm_ref, num2_ref, den2_ref, xout_ref):
    xm = xm_ref[...]
    den = den2_ref[...] + 1e-16
    num2 = num2_ref[...]
    cols = []
    for c in range(3):
        ratio = num2[:, c * _NH:(c + 1) * _NH] / den
        d = jnp.sum(ratio, axis=-1, keepdims=True) * (1.0 / _NH)
        cols.append(xm[:, c:c + 1] + d * xm[:, 3:4])
    cols.append(jnp.zeros((xm.shape[0], 13), _F32))
    xout_ref[...] = jnp.concatenate(cols, axis=1)


def _edge1_body(b0_ref, nw_ref, dst_ref, hs_ref, xms_ref,
                h_full, q_full, xm_full,
                kW1r, kW1d, kW1s, kb1, kg, kbe, kW2, kb2,
                vW1r, vW1d, vW1s, vb1, vg, vbe, vW2, vb2,
                ewW, ewb, num_ref, den_ref, hd_s, qd_s, xmd_s, *, E_static):
    i = pl.program_id(0)

    @pl.when(i == 0)
    def _init():
        num_ref[...] = jnp.zeros_like(num_ref)
        den_ref[...] = jnp.zeros_like(den_ref)

    dst = dst_ref[0, 0, :]
    hs = hs_ref[...]
    b0 = b0_ref[i]
    hd_s[...] = jnp.zeros_like(hd_s)
    qd_s[...] = jnp.zeros_like(qd_s)
    xmd_s[...] = jnp.zeros_like(xmd_s)

    def gbody(w, carry):
        base = b0 + w * _W
        oh = (dst[:, None] ==
              base + lax.broadcasted_iota(jnp.int32, (_T, _W), 1)).astype(_F32)
        hd_s[...] += _mm(oh, h_full[pl.ds(base, _W), :])
        qd_s[...] += _mm(oh, q_full[pl.ds(base, _W), :])
        xmd_s[...] += _mm(oh, xm_full[pl.ds(base, _W), :][:, :16])
        return carry

    lax.fori_loop(0, nw_ref[i], gbody, 0)
    hd = hd_s[...]
    qd = qd_s[...]
    r80, _ = _r_feat(xms_ref[...][:, :16], xmd_s[...])

    yk = _mm(r80, kW1r[...]) + _mm(hd, kW1d[...]) + _mm(hs, kW1s[...]) + kb1[...]
    k = _mm(_ln_relu(yk, kg[...], kbe[...]), kW2[...]) + kb2[...]
    yv = _mm(r80, vW1r[...]) + _mm(hd, vW1d[...]) + _mm(hs, vW1s[...]) + vb1[...]
    v = _mm(_ln_relu(yv, vg[...], vbe[...]), vW2[...]) + vb2[...]
    v = v * jax.nn.sigmoid(_mm(r80, ewW[...]) + ewb[...])

    S = _head_sum_mat()
    l = _mm(qd * k, S) * (1.0 / np.sqrt(_HD))
    gidx = i * _T + lax.broadcasted_iota(jnp.int32, (_T, 1), 0)
    valid = (gidx < E_static).astype(_F32)
    e = jnp.exp(jnp.clip(l, -50.0, 50.0)) * valid
    ev = _mm(e, _head_rep_mat()) * v

    def wbody(w, carry):
        base = b0 + w * _W
        oh = (dst[:, None] ==
              base + lax.broadcasted_iota(jnp.int32, (_T, _W), 1)).astype(_F32)
        num_ref[pl.ds(base, _W), :] += _dotT(oh, ev)
        den_ref[pl.ds(base, _W), :] += _dotT(oh, e)
        return carry

    lax.fori_loop(0, nw_ref[i], wbody, 0)


def _edge2_body(b0_ref, nw_ref, dst_ref, hs_ref, xms_ref,
                h_full, q_full, xm_full,
                kW1r, kW1d, kW1s, kb1, kg, kbe, kW2, kb2,
                vW1r, vW1d, vW1s, vb1, vg, vbe, vW2, vb2,
                ewW, ewb, num_ref, den_ref, hd_s, qd_s, xmd_s, *, E_static):
    i = pl.program_id(0)

    @pl.when(i == 0)
    def _init():
        num_ref[...] = jnp.zeros_like(num_ref)
        den_ref[...] = jnp.zeros_like(den_ref)

    dst = dst_ref[0, 0, :]
    hs = hs_ref[...]
    b0 = b0_ref[i]
    hd_s[...] = jnp.zeros_like(hd_s)
    qd_s[...] = jnp.zeros_like(qd_s)
    xmd_s[...] = jnp.zeros_like(xmd_s)

    def gbody(w, carry):
        base = b0 + w * _W
        oh = (dst[:, None] ==
              base + lax.broadcasted_iota(jnp.int32, (_T, _W), 1)).astype(_F32)
        hd_s[...] += _mm(oh, h_full[pl.ds(base, _W), :])
        qd_s[...] += _mm(oh, q_full[pl.ds(base, _W), :])
        xmd_s[...] += _mm(oh, xm_full[pl.ds(base, _W), :][:, :16])
        return carry

    lax.fori_loop(0, nw_ref[i], gbody, 0)
    hd = hd_s[...]
    qd = qd_s[...]
    r80, relm = _r_feat(xms_ref[...][:, :16], xmd_s[...])

    yk = _mm(r80, kW1r[...]) + _mm(hd, kW1d[...]) + _mm(hs, kW1s[...]) + kb1[...]
    k = _mm(_ln_relu(yk, kg[...], kbe[...]), kW2[...]) + kb2[...]
    yv = _mm(r80, vW1r[...]) + _mm(hd, vW1d[...]) + _mm(hs, vW1s[...]) + vb1[...]
    xv = _mm(_ln_relu(yv, vg[...], vbe[...]), vW2[...]) + vb2[...]
    xv = xv * jax.nn.sigmoid(_mm(r80, ewW[...]) + ewb[...])

    S = _head_sum_mat()
    l = _mm(qd * k, S) * (1.0 / np.sqrt(_HD))
    gidx = i * _T + lax.broadcasted_iota(jnp.int32, (_T, 1), 0)
    valid = (gidx < E_static).astype(_F32)
    e = jnp.exp(jnp.clip(l, -50.0, 50.0)) * valid
    w16 = e * xv
    w48 = jnp.concatenate([w16 * relm[:, c:c + 1] for c in range(3)], axis=1)

    def wbody(w, carry):
        base = b0 + w * _W
        oh = (dst[:, None] ==
              base + lax.broadcasted_iota(jnp.int32, (_T, _W), 1)).astype(_F32)
        num_ref[pl.ds(base, _W), :] += _dotT(oh, w48)
        den_ref[pl.ds(base, _W), :] += _dotT(oh, e)
        return carry

    lax.fori_loop(0, nw_ref[i], wbody, 0)


def _full(shape):
    return pl.BlockSpec(shape, lambda i, b0, nw: tuple(0 for _ in shape))


def _edge_call(body, b0, nw, dst3, hs, xms, hp, qp, xmp, wts, E, Npd, dout):
    nT = dst3.shape[0]
    espec = [
        pl.BlockSpec((1, 1, _T), lambda i, b0, nw: (i, 0, 0)),
        pl.BlockSpec((_T, 128), lambda i, b0, nw: (i, 0)),
        pl.BlockSpec((_T, 128), lambda i, b0, nw: (i, 0)),
        _full((Npd, 128)),
        _full((Npd, 128)),
        _full((Npd, 128)),
    ]
    wspec = [_full(w.shape) for w in wts]
    grid_spec = pltpu.PrefetchScalarGridSpec(
        num_scalar_prefetch=2,
        grid=(nT,),
        in_specs=espec + wspec,
        out_specs=[
            pl.BlockSpec((Npd, dout), lambda i, b0, nw: (0, 0)),
            pl.BlockSpec((Npd, 16), lambda i, b0, nw: (0, 0)),
        ],
        scratch_shapes=[
            pltpu.VMEM((_T, 128), _F32),
            pltpu.VMEM((_T, 128), _F32),
            pltpu.VMEM((_T, 16), _F32),
        ],
    )
    return pl.pallas_call(
        functools.partial(body, E_static=E),
        grid_spec=grid_spec,
        out_shape=[
            jax.ShapeDtypeStruct((Npd, dout), _F32),
            jax.ShapeDtypeStruct((Npd, 16), _F32),
        ],
    )(b0, nw, dst3, hs, xms, hp, qp, xmp, *wts)


def _node_call(body, ins, outs_dims, N, Tn):
    specs = [pl.BlockSpec((Tn, a.shape[1]), lambda i: (i, 0)) if a.shape[0] == N
             else pl.BlockSpec(a.shape, lambda i: (0, 0)) for a in ins]
    return pl.pallas_call(
        body,
        grid=(N // Tn,),
        in_specs=specs,
        out_specs=[pl.BlockSpec((Tn, d), lambda i: (i, 0)) for d in outs_dims],
        out_shape=[jax.ShapeDtypeStruct((N, d), _F32) for d in outs_dims],
    )(*ins)


def _mlp_wts(p, split=None):
    W1 = p['W1']
    row = lambda a: a.reshape(1, -1)
    if split is None:
        parts = [W1]
    else:
        cuts = [0] + list(split) + [W1.shape[0]]
        parts = [W1[cuts[j]:cuts[j + 1]] for j in range(len(cuts) - 1)]
    return parts + [row(p['b1']), row(p['g']), row(p['be']), p['W2'],
                    row(p['b2'])]


def kernel(h, x, params, mask_ligand, edge_index):
    N, D = h.shape
    E = edge_index.shape[1]
    Tn = 2000 if N % 2000 == 0 else N
    Npd = N + _W

    order = jnp.argsort(edge_index[1])
    dsts = edge_index[1][order]
    srcs = edge_index[0][order]
    quantum = int(np.lcm(32 * _C, _T))
    E_pad = int(-(-E // quantum) * quantum)
    dsts_p = jnp.pad(dsts, (0, E_pad - E), constant_values=N - 1)
    srcs_p = jnp.pad(srcs, (0, E_pad - E))
    nT = E_pad // _T
    dmat = dsts_p.reshape(nT, _T)
    b0 = dmat[:, 0].astype(jnp.int32)
    nw = ((dmat[:, -1] - dmat[:, 0]) // _W + 1).astype(jnp.int32)
    dst3 = dsts_p.reshape(nT, 1, _T)

    mlf = mask_ligand.astype(_F32)[:, None]
    xm = jnp.concatenate([x, mlf, jnp.zeros((N, 124), _F32)], axis=1)

    q = _node_call(_qmlp_body, [h] + _mlp_wts(params['x2h_q']), [D], N, Tn)[0]
    hs = _sc_gather(h, srcs_p)
    xms = _sc_gather(xm, srcs_p)
    pad_n = lambda a: jnp.pad(a, ((0, Npd - N), (0, 0)))
    hp, qp, xmp = pad_n(h), pad_n(q), pad_n(xm)

    ew1 = jnp.broadcast_to(params['x2h_ew_W'], (4 * _NG, 128))
    eb1 = jnp.broadcast_to(params['x2h_ew_b'].reshape(1, 1), (1, 128))
    wts1 = (_mlp_wts(params['x2h_k'], (4 * _NG, 4 * _NG + D))
            + _mlp_wts(params['x2h_v'], (4 * _NG, 4 * _NG + D))
            + [ew1, eb1])
    num, den = _edge_call(_edge1_body, b0, nw, dst3, hs, xms, hp, qp, xmp,
                          wts1, E, Npd, 128)

    h_out, q2 = _node_call(
        _h_update_body,
        [num[:N], den[:N], h]
        + _mlp_wts(params['x2h_out'], (D,))
        + _mlp_wts(params['h2x_q']),
        [D, D], N, Tn)

    hs2 = _sc_gather(h_out, srcs_p)
    hp2, q2p = pad_n(h_out), pad_n(q2)

    ew2 = jnp.broadcast_to(params['h2x_ew_W'], (4 * _NG, 16))
    eb2 = jnp.broadcast_to(params['h2x_ew_b'].reshape(1, 1), (1, 16))
    wts2 = (_mlp_wts(params['h2x_k'], (4 * _NG, 4 * _NG + D))
            + _mlp_wts(params['h2x_v'], (4 * _NG, 4 * _NG + D))
            + [ew2, eb2])
    num2, den2 = _edge_call(_edge2_body, b0, nw, dst3, hs2, xms, hp2, q2p, xmp,
                            wts2, E, Npd, 48)

    xout = _node_call(_x_update_body, [xm, num2[:N], den2[:N]], [16], N, Tn)[0]
    return h_out, xout[:, :3]
